# Initial kernel scaffold; baseline (speedup 1.0000x reference)
import jax
import jax.numpy as jnp
from jax.experimental import pallas as pl


def kernel(hidden_states, ln_g, ln_b, W_fc1, b_fc1, W_fc2, b_fc2, W_router, b_router, We1, be1, We2, be2):
    raise NotImplementedError("write your pallas kernel here")


# speedup vs baseline: 1.0551x; 1.0551x over previous
import jax
import jax.numpy as jnp
from jax.experimental import pallas as pl

_B, _S, _D = 1, 2048, 768
_DFF = 3072
_E = 8
_DH = 768
_EPS = 1e-5
_TT = 256


def _top2(p):
    NEG = jnp.float32(-1e30)
    m1 = jnp.full(p.shape[:1] + (1,), NEG, jnp.float32)
    i1 = jnp.zeros(p.shape[:1] + (1,), jnp.int32)
    for e in range(_E):
        pe = p[:, e : e + 1]
        upd = pe > m1
        m1 = jnp.where(upd, pe, m1)
        i1 = jnp.where(upd, e, i1)
    m2 = jnp.full(p.shape[:1] + (1,), NEG, jnp.float32)
    i2 = jnp.zeros(p.shape[:1] + (1,), jnp.int32)
    for e in range(_E):
        pe = jnp.where(i1 == e, NEG, p[:, e : e + 1])
        upd = pe > m2
        m2 = jnp.where(upd, pe, m2)
        i2 = jnp.where(upd, e, i2)
    return m1, i1, m2, i2


def _dense_body(x_ref, g_ref, b_ref, w1_ref, b1_ref, w2_ref, b2_ref,
                wr_ref, br_ref, we1_ref, be1_ref, we2_ref, be2_ref, o_ref):
    x = x_ref[...]
    mu = jnp.mean(x, axis=-1, keepdims=True)
    var = jnp.mean((x - mu) ** 2, axis=-1, keepdims=True)
    h = (x - mu) / jnp.sqrt(var + _EPS) * g_ref[...] + b_ref[...]
    hb = h.astype(jnp.bfloat16)

    t1 = jnp.dot(hb, w1_ref[...], preferred_element_type=jnp.float32) + b1_ref[...]
    s = jax.nn.gelu(t1)
    sh = jnp.dot(s.astype(jnp.bfloat16), w2_ref[...],
                 preferred_element_type=jnp.float32) + b2_ref[...]

    logits = jnp.dot(hb, wr_ref[...],
                     preferred_element_type=jnp.float32) + br_ref[...]
    m = jnp.max(logits, axis=-1, keepdims=True)
    p = jnp.exp(logits - m)
    p = p / jnp.sum(p, axis=-1, keepdims=True)
    m1, i1, m2, i2 = _top2(p)
    wsum = m1 + m2
    w1 = m1 / wsum
    w2 = m2 / wsum

    acc = x + sh
    for e in range(_E):
        t = jnp.dot(hb, we1_ref[e], preferred_element_type=jnp.float32) + be1_ref[e]
        tg = jax.nn.gelu(t)
        o = jnp.dot(tg.astype(jnp.bfloat16), we2_ref[e],
                    preferred_element_type=jnp.float32) + be2_ref[e]
        ge = jnp.where(i1 == e, w1, 0.0) + jnp.where(i2 == e, w2, 0.0)
        acc = acc + ge * o
    o_ref[...] = acc


def kernel(hidden_states, ln_g, ln_b, W_fc1, b_fc1, W_fc2, b_fc2,
           W_router, b_router, We1, be1, We2, be2):
    xf = hidden_states.reshape(_S, _D)
    wrb = W_router.astype(jnp.bfloat16)
    w1b = W_fc1.astype(jnp.bfloat16)
    w2b = W_fc2.astype(jnp.bfloat16)
    we1b = We1.astype(jnp.bfloat16)
    we2b = We2.astype(jnp.bfloat16)

    full = lambda shape: pl.BlockSpec(shape, lambda i: (0,) * len(shape))
    out = pl.pallas_call(
        _dense_body,
        grid=(_S // _TT,),
        in_specs=[
            pl.BlockSpec((_TT, _D), lambda i: (i, 0)),
            full((_D,)), full((_D,)),
            full((_D, _DFF)), full((_DFF,)),
            full((_DFF, _D)), full((_D,)),
            full((_D, _E)), full((_E,)),
            full((_E, _D, _DH)), full((_E, _DH)),
            full((_E, _DH, _D)), full((_E, _D)),
        ],
        out_specs=pl.BlockSpec((_TT, _D), lambda i: (i, 0)),
        out_shape=jax.ShapeDtypeStruct((_S, _D), jnp.float32),
    )(xf, ln_g, ln_b, w1b, b_fc1, w2b, b_fc2,
      wrb, b_router, we1b, be1, we2b, be2)
    return out.reshape(_B, _S, _D)
